# Initial kernel scaffold; baseline (speedup 1.0000x reference)
import jax
import jax.numpy as jnp
from jax.experimental import pallas as pl


def kernel(mu, alpha, gamma, ti, mi, T):
    raise NotImplementedError("write your pallas kernel here")


# speedup vs baseline: 274.0888x; 274.0888x over previous
import functools

import jax
import jax.numpy as jnp
from jax.experimental import pallas as pl
from jax.experimental.pallas import tpu as pltpu

_BIG = 1e9


def _hawkes_body(N, B, K, M,
                 t_col_ref, t_row_ref, mi_col_ref, alpha_ref, mu_ref,
                 gamma_ref, tf_ref, anch_ref, panch_ref,
                 out_ref, carry_ref):
    c = pl.program_id(0)

    tc = t_col_ref[0]
    tr = t_row_ref[0]
    mic = mi_col_ref[0]
    Tf = tf_ref[0, 0]

    @pl.when(c == 0)
    def _init():
        out_ref[0, 0] = Tf * jnp.sum(mu_ref[...])
        carry_ref[...] = jnp.zeros_like(carry_ref)

    gidx = jax.lax.broadcasted_iota(jnp.int32, (B, 1), 0) + c * B
    valid = gidx < N

    miota = jax.lax.broadcasted_iota(jnp.int32, (B, M), 1)
    P = jnp.where((mic == miota) & valid, 1.0, 0.0).astype(jnp.float32)

    ii = jax.lax.broadcasted_iota(jnp.int32, (B, B), 0)
    jj = jax.lax.broadcasted_iota(jnp.int32, (B, B), 1)
    delta = jnp.where(ii > jj, tc - tr, _BIG)

    b_prev = panch_ref[0, c]
    b_new = anch_ref[0, c]

    exc = jnp.zeros((B, 1), jnp.float32)
    step_sum = jnp.float32(0.0)
    for k in range(K):
        gk = gamma_ref[0, k]
        Ek = jnp.exp(-gk * delta)
        Wk = jnp.dot(Ek, P, preferred_element_type=jnp.float32)
        dcross = jnp.exp(-gk * (tc - b_prev))
        Ck = carry_ref[k:k + 1, :]
        Xk = Wk + dcross * Ck
        alpha_k = alpha_ref[k]
        Yk = jnp.dot(Xk, alpha_k, preferred_element_type=jnp.float32)
        exc = exc + gk * jnp.sum(Yk * P, axis=1, keepdims=True)

        asum_k = jnp.sum(alpha_k, axis=1, keepdims=True)
        ck = 1.0 - jnp.exp(-gk * (Tf - tc))
        step_sum += jnp.sum(jnp.dot(P, asum_k, preferred_element_type=jnp.float32) * ck)

        fj = jnp.exp(-gk * (b_new - tr))
        Gk = jnp.dot(fj, P, preferred_element_type=jnp.float32)
        dblk = jnp.exp(-gk * (b_new - b_prev))
        carry_ref[k:k + 1, :] = dblk * Ck + Gk

    mu_i = jnp.sum(mu_ref[...] * P, axis=1, keepdims=True)
    lam = mu_i + exc
    lam_safe = jnp.where(valid, lam, 1.0)
    step_sum += -jnp.sum(jnp.log(lam_safe))

    out_ref[0, 0] += step_sum


def kernel(mu, alpha, gamma, ti, mi, T):
    N = ti.shape[1]
    M = mu.shape[0]
    K = gamma.shape[0]
    B = 256
    C = -(-N // B)
    NP = C * B
    pad = NP - N

    t = ti.reshape(N).astype(jnp.float32)
    if pad:
        t_pad = jnp.concatenate([t, jnp.broadcast_to(t[N - 1], (pad,))])
        mi_pad = jnp.concatenate([mi.astype(jnp.int32),
                                  jnp.zeros((pad,), jnp.int32)])
    else:
        t_pad = t
        mi_pad = mi.astype(jnp.int32)

    t_col = t_pad.reshape(C, B, 1)
    t_row = t_pad.reshape(C, 1, B)
    mi_col = mi_pad.reshape(C, B, 1)
    anchors = t_pad[B - 1::B].reshape(1, C)
    prev_anchors = jnp.concatenate(
        [jnp.zeros((1, 1), jnp.float32), anchors[:, :-1]], axis=1)
    gamma2 = gamma.reshape(1, K).astype(jnp.float32)
    mu2 = mu.reshape(1, M).astype(jnp.float32)
    alpha3 = alpha.astype(jnp.float32)
    Tf = jnp.asarray(T, jnp.float32).reshape(1, 1)

    body = functools.partial(_hawkes_body, N, B, K, M)
    out = pl.pallas_call(
        body,
        grid=(C,),
        in_specs=[
            pl.BlockSpec((1, B, 1), lambda c: (c, 0, 0)),
            pl.BlockSpec((1, 1, B), lambda c: (c, 0, 0)),
            pl.BlockSpec((1, B, 1), lambda c: (c, 0, 0)),
            pl.BlockSpec((K, M, M), lambda c: (0, 0, 0)),
            pl.BlockSpec((1, M), lambda c: (0, 0)),
            pl.BlockSpec(memory_space=pltpu.SMEM),
            pl.BlockSpec(memory_space=pltpu.SMEM),
            pl.BlockSpec(memory_space=pltpu.SMEM),
            pl.BlockSpec(memory_space=pltpu.SMEM),
        ],
        out_specs=pl.BlockSpec(memory_space=pltpu.SMEM),
        out_shape=jax.ShapeDtypeStruct((1, 1), jnp.float32),
        scratch_shapes=[pltpu.VMEM((K, M), jnp.float32)],
    )(t_col, t_row, mi_col, alpha3, mu2, gamma2, Tf, anchors, prev_anchors)
    return out[0, 0] / jnp.float32(N)
